# baseline (speedup 1.0000x reference)
import functools

import jax
import jax.numpy as jnp
import numpy as np
from jax import lax
from jax.experimental import pallas as pl
from jax.experimental.pallas import tpu as pltpu
from jax.experimental.pallas import tpu_sc as plsc

B, L, D = 4096, 200, 64
N_IDX = B * L
IDX_COLS = 128
IDX_ROWS = N_IDX // IDX_COLS
NC, NS = 2, 16
NW = NC * NS
W_IDX_ROWS = IDX_ROWS // NW
CHUNK_IR = 4
CHUNK_ROWS = CHUNK_IR * IDX_COLS
N_CHUNKS = W_IDX_ROWS // CHUNK_IR

MAG = float(np.float32(5.0) / np.sqrt(np.float32(L * D)))

N_ELEM = N_IDX * D
ROWS128 = N_ELEM // 128
BLK = 2048


def _sc_gather(table, idx2d):
    mesh = plsc.VectorSubcoreMesh(core_axis_name="c", subcore_axis_name="s")

    @functools.partial(
        pl.kernel,
        mesh=mesh,
        compiler_params=pltpu.CompilerParams(use_tc_tiling_on_sc=False),
        out_type=jax.ShapeDtypeStruct((N_IDX, D), jnp.float32),
        scratch_types=[
            pltpu.VMEM((CHUNK_IR, IDX_COLS), jnp.int32),
            pltpu.VMEM((CHUNK_ROWS, D), jnp.float32),
            pltpu.SemaphoreType.DMA,
        ],
    )
    def k(table_hbm, idx_hbm, out_hbm, idx_v, rows_v, sem):
        wid = lax.axis_index("s") * NC + lax.axis_index("c")
        base_ir = wid * W_IDX_ROWS

        def body(c, _):
            ir = base_ir + c * CHUNK_IR
            pltpu.sync_copy(idx_hbm.at[pl.ds(ir, CHUNK_IR)], idx_v)
            cps = [
                pltpu.async_copy(
                    table_hbm.at[idx_v.at[j]],
                    rows_v.at[pl.ds(j * IDX_COLS, IDX_COLS)],
                    sem,
                )
                for j in range(CHUNK_IR)
            ]
            for cp in cps:
                cp.wait()
            pltpu.sync_copy(rows_v, out_hbm.at[pl.ds(ir * IDX_COLS, CHUNK_ROWS)])
            return _

        lax.fori_loop(0, N_CHUNKS, body, None)

    return k(table, idx2d)


def _noise_add_body(x_ref, o_ref):
    pid = pl.program_id(0)
    base = (pid * (BLK * 128)).astype(jnp.uint32)
    it = (
        lax.broadcasted_iota(jnp.int32, (BLK, 128), 0) * 128
        + lax.broadcasted_iota(jnp.int32, (BLK, 128), 1)
    ).astype(jnp.uint32)
    x1 = base + it
    x0 = jnp.zeros_like(x1)

    k0 = jnp.uint32(0)
    k1 = jnp.uint32(42)
    k2 = jnp.uint32(0x1BD11BDA ^ 42)

    def rotl(v, r):
        return (v << jnp.uint32(r)) | (v >> jnp.uint32(32 - r))

    xa = x0 + k0
    xb = x1 + k1
    ks = (k0, k1, k2)
    rots = ((13, 15, 26, 6), (17, 29, 16, 24))
    for i in range(5):
        for r in rots[0]:
            xa = xa + xb
            xb = rotl(xb, r)
            xb = xa ^ xb
        xa = xa + ks[1]
        xb = xb + (ks[2] + jnp.uint32(i + 1))
        ks = (ks[1], ks[2], ks[0])
        rots = (rots[1], rots[0])
    bits = xa ^ xb

    uf = lax.bitcast_convert_type(
        (bits >> jnp.uint32(9)) | jnp.uint32(0x3F800000), jnp.float32
    )
    u = uf - jnp.float32(1.0)
    r2 = u * jnp.float32(2.0) - jnp.float32(1.0)
    o_ref[...] = x_ref[...] + r2 * jnp.float32(MAG)


def _tc_noise_add(e128):
    return pl.pallas_call(
        _noise_add_body,
        grid=(ROWS128 // BLK,),
        in_specs=[pl.BlockSpec((BLK, 128), lambda i: (i, 0))],
        out_specs=pl.BlockSpec((BLK, 128), lambda i: (i, 0)),
        out_shape=jax.ShapeDtypeStruct((ROWS128, 128), jnp.float32),
    )(e128)


def kernel(input_ids, table):
    ids = input_ids.reshape(IDX_ROWS, IDX_COLS).astype(jnp.int32)
    embeds = _sc_gather(table, ids)
    out128 = _tc_noise_add(embeds.reshape(ROWS128, 128))
    return out128.reshape(B, L, D)
